# baseline (speedup 1.0000x reference)
import functools

import jax
import jax.numpy as jnp
from jax import lax
from jax.experimental import pallas as pl
from jax.experimental.pallas import tpu as pltpu
from jax.experimental.pallas import tpu_sc as plsc

_SIZE = 224
_NC, _NS = 2, 16
_NW = _NC * _NS
_C = 192
_CPW = _C // _NW
_NVEC = _SIZE // 16


def _crop_body(img, ind16, out, c_out, ind_v, c_v, ibuf, obuf):
    wid = lax.axis_index("s") * _NC + lax.axis_index("c")
    pltpu.sync_copy(ind16, ind_v)
    iv = ind_v[...]
    x = iv[0]
    y = iv[1]
    x8 = pl.multiple_of((x // 8) * 8, 8)
    r0 = x - x8

    @pl.when(wid == 0)
    def _():
        c_v[...] = iv + _SIZE // 2
        pltpu.sync_copy(c_v.at[pl.ds(0, 2)], c_out)

    base = wid * _CPW
    for i in range(_CPW):
        pltpu.sync_copy(
            img.at[pl.ds(base + i, 1), pl.ds(y, _SIZE), pl.ds(x8, _SIZE + 8)],
            ibuf,
        )

        def row(j, carry):
            for k in range(_NVEC):
                obuf[0, j, pl.ds(16 * k, 16)] = ibuf[0, j, pl.ds(r0 + 16 * k, 16)]
            return carry

        lax.fori_loop(0, _SIZE, row, 0)
        pltpu.sync_copy(obuf, out.at[pl.ds(base + i, 1)])


@jax.jit
def _crop_call(img, ind16):
    mesh = plsc.VectorSubcoreMesh(
        core_axis_name="c", subcore_axis_name="s", num_cores=_NC, num_subcores=_NS
    )
    return pl.kernel(
        _crop_body,
        out_type=[
            jax.ShapeDtypeStruct((_C, _SIZE, _SIZE), jnp.float32),
            jax.ShapeDtypeStruct((2,), jnp.int32),
        ],
        mesh=mesh,
        scratch_types=[
            pltpu.VMEM((16,), jnp.int32),
            pltpu.VMEM((16,), jnp.int32),
            pltpu.VMEM((1, _SIZE, _SIZE + 8), jnp.float32),
            pltpu.VMEM((1, _SIZE, _SIZE), jnp.float32),
        ],
        compiler_params=pltpu.CompilerParams(use_tc_tiling_on_sc=False),
    )(img, ind16)


def kernel(img, pmap, ind):
    ind16 = jnp.zeros((16,), jnp.int32).at[:2].set(ind[0])
    cropped, c = _crop_call(img, ind16)
    return cropped, c


# speedup vs baseline: 2.9011x; 2.9011x over previous
import functools

import jax
import jax.numpy as jnp
from jax import lax
from jax.experimental import pallas as pl
from jax.experimental.pallas import tpu as pltpu
from jax.experimental.pallas import tpu_sc as plsc

_SIZE = 224
_NC, _NS = 2, 16
_NW = _NC * _NS
_C = 192
_CPW = _C // _NW
_NVEC = _SIZE // 16
_ROWS = 56
_WROWS = _ROWS + 8
_WCOLS = 384
_NCHUNK = _SIZE // _ROWS
_NT = _CPW * _NCHUNK


def _crop_body(img, ind16, out, c_out, ind_v, c_v, ib0, ib1, ob0, ob1, isem, osem):
    wid = lax.axis_index("s") * _NC + lax.axis_index("c")
    pltpu.sync_copy(ind16, ind_v)
    iv = ind_v[...]
    x = iv[0]
    y = iv[1]
    y8 = pl.multiple_of((y // 8) * 8, 8)
    x128 = pl.multiple_of((x // 128) * 128, 128)
    ry = y - y8
    rx = x - x128

    @pl.when(wid == 0)
    def _():
        c_v[...] = iv + _SIZE // 2
        pltpu.sync_copy(c_v.at[pl.ds(0, 2)], c_out)

    ibufs = [ib0, ib1]
    obufs = [ob0, ob1]
    base = wid * _CPW

    def start_in(t, s):
        ch = base + t // _NCHUNK
        q = t % _NCHUNK
        pltpu.make_async_copy(
            img.at[
                pl.ds(ch, 1),
                pl.ds(y8 + _ROWS * q, _WROWS),
                pl.ds(x128, _WCOLS),
            ],
            ibufs[s],
            isem.at[s],
        ).start()

    def wait_in(t, s):
        ch = base + t // _NCHUNK
        q = t % _NCHUNK
        pltpu.make_async_copy(
            img.at[
                pl.ds(ch, 1),
                pl.ds(y8 + _ROWS * q, _WROWS),
                pl.ds(x128, _WCOLS),
            ],
            ibufs[s],
            isem.at[s],
        ).wait()

    def start_out(t, s):
        ch = base + t // _NCHUNK
        q = t % _NCHUNK
        pltpu.make_async_copy(
            obufs[s],
            out.at[pl.ds(ch, 1), pl.ds(_ROWS * q, _ROWS)],
            osem.at[s],
        ).start()

    def wait_out(t, s):
        ch = base + t // _NCHUNK
        q = t % _NCHUNK
        pltpu.make_async_copy(
            obufs[s],
            out.at[pl.ds(ch, 1), pl.ds(_ROWS * q, _ROWS)],
            osem.at[s],
        ).wait()

    iota = lax.iota(jnp.int32, 16)
    d0 = jnp.zeros((16,), jnp.int32)
    d2 = [rx + 16 * k + iota for k in range(_NVEC)]

    def compute(s):
        ib = ibufs[s]
        ob = obufs[s]

        def row(j, carry):
            d1 = jnp.full((16,), ry + j, jnp.int32)
            for k in range(_NVEC):
                v = plsc.load_gather(ib, [d0, d1, d2[k]])
                ob[0, j, pl.ds(16 * k, 16)] = v
            return carry

        lax.fori_loop(0, _ROWS, row, 0)

    start_in(0, 0)
    start_in(1, 1)
    for t in range(_NT):
        s = t % 2
        wait_in(t, s)
        if t >= 2:
            wait_out(t - 2, s)
        compute(s)
        start_out(t, s)
        if t + 2 < _NT:
            start_in(t + 2, s)
    wait_out(_NT - 2, _NT % 2)
    wait_out(_NT - 1, (_NT + 1) % 2)


@jax.jit
def _crop_call(img, ind16):
    mesh = plsc.VectorSubcoreMesh(
        core_axis_name="c", subcore_axis_name="s", num_cores=_NC, num_subcores=_NS
    )
    return pl.kernel(
        _crop_body,
        out_type=[
            jax.ShapeDtypeStruct((_C, _SIZE, _SIZE), jnp.float32),
            jax.ShapeDtypeStruct((2,), jnp.int32),
        ],
        mesh=mesh,
        scratch_types=[
            pltpu.VMEM((16,), jnp.int32),
            pltpu.VMEM((16,), jnp.int32),
            pltpu.VMEM((1, _WROWS, _WCOLS), jnp.float32),
            pltpu.VMEM((1, _WROWS, _WCOLS), jnp.float32),
            pltpu.VMEM((1, _ROWS, _SIZE), jnp.float32),
            pltpu.VMEM((1, _ROWS, _SIZE), jnp.float32),
            pltpu.SemaphoreType.DMA((2,)),
            pltpu.SemaphoreType.DMA((2,)),
        ],
        compiler_params=pltpu.CompilerParams(
            use_tc_tiling_on_sc=True, needs_layout_passes=False
        ),
    )(img, ind16)


def kernel(img, pmap, ind):
    ind16 = jnp.zeros((16,), jnp.int32).at[:2].set(ind[0])
    cropped, c = _crop_call(img, ind16)
    return cropped, c


# speedup vs baseline: 4.1800x; 1.4408x over previous
import functools

import jax
import jax.numpy as jnp
from jax import lax
from jax.experimental import pallas as pl
from jax.experimental.pallas import tpu as pltpu
from jax.experimental.pallas import tpu_sc as plsc

_SIZE = 224
_NC, _NS = 2, 16
_NW = _NC * _NS
_C = 192
_CPW = _C // _NW
_NVEC = _SIZE // 16
_ROWS = 56
_WROWS = _ROWS + 8
_WCOLS = 384
_NCHUNK = _SIZE // _ROWS
_NT = _CPW * _NCHUNK


def _crop_body(img, ind16, out, c_out, ind_v, c_v, ib0, ib1, ob0, ob1, isem, osem):
    wid = lax.axis_index("s") * _NC + lax.axis_index("c")
    pltpu.sync_copy(ind16, ind_v)
    iv = ind_v[...]
    x = iv[0]
    y = iv[1]
    y8 = pl.multiple_of((y // 8) * 8, 8)
    x128 = pl.multiple_of((x // 128) * 128, 128)
    ry = y - y8
    rx = x - x128

    @pl.when(wid == 0)
    def _():
        c_v[...] = iv + _SIZE // 2
        pltpu.sync_copy(c_v.at[pl.ds(0, 2)], c_out)

    ibufs = [ib0, ib1]
    obufs = [ob0, ob1]
    base = wid * _CPW

    def start_in(t, s):
        ch = base + t // _NCHUNK
        q = t % _NCHUNK
        pltpu.make_async_copy(
            img.at[
                pl.ds(ch, 1),
                pl.ds(y8 + _ROWS * q, _WROWS),
                pl.ds(x128, _WCOLS),
            ],
            ibufs[s],
            isem.at[s],
        ).start()

    def wait_in(t, s):
        ch = base + t // _NCHUNK
        q = t % _NCHUNK
        pltpu.make_async_copy(
            img.at[
                pl.ds(ch, 1),
                pl.ds(y8 + _ROWS * q, _WROWS),
                pl.ds(x128, _WCOLS),
            ],
            ibufs[s],
            isem.at[s],
        ).wait()

    def start_out(t, s):
        ch = base + t // _NCHUNK
        q = t % _NCHUNK
        pltpu.make_async_copy(
            obufs[s],
            out.at[pl.ds(ch, 1), pl.ds(_ROWS * q, _ROWS)],
            osem.at[s],
        ).start()

    def wait_out(t, s):
        ch = base + t // _NCHUNK
        q = t % _NCHUNK
        pltpu.make_async_copy(
            obufs[s],
            out.at[pl.ds(ch, 1), pl.ds(_ROWS * q, _ROWS)],
            osem.at[s],
        ).wait()

    iota = lax.iota(jnp.int32, 16)
    d0 = jnp.zeros((16,), jnp.int32)
    d2 = [rx + 16 * k + iota for k in range(_NVEC)]

    def compute(s):
        ib = ibufs[s]
        ob = obufs[s]

        def row(j, carry):
            j0 = 2 * j
            d1a = jnp.full((16,), ry + j0, jnp.int32)
            d1b = d1a + 1
            va = [plsc.load_gather(ib, [d0, d1a, d2[k]]) for k in range(_NVEC)]
            vb = [plsc.load_gather(ib, [d0, d1b, d2[k]]) for k in range(_NVEC)]
            for k in range(_NVEC):
                ob[0, j0, pl.ds(16 * k, 16)] = va[k]
            for k in range(_NVEC):
                ob[0, j0 + 1, pl.ds(16 * k, 16)] = vb[k]
            return carry

        lax.fori_loop(0, _ROWS // 2, row, 0)

    start_in(0, 0)
    start_in(1, 1)
    for t in range(_NT):
        s = t % 2
        wait_in(t, s)
        if t >= 2:
            wait_out(t - 2, s)
        compute(s)
        start_out(t, s)
        if t + 2 < _NT:
            start_in(t + 2, s)
    wait_out(_NT - 2, _NT % 2)
    wait_out(_NT - 1, (_NT + 1) % 2)


@jax.jit
def _crop_call(img, ind16):
    mesh = plsc.VectorSubcoreMesh(
        core_axis_name="c", subcore_axis_name="s", num_cores=_NC, num_subcores=_NS
    )
    return pl.kernel(
        _crop_body,
        out_type=[
            jax.ShapeDtypeStruct((_C, _SIZE, _SIZE), jnp.float32),
            jax.ShapeDtypeStruct((2,), jnp.int32),
        ],
        mesh=mesh,
        scratch_types=[
            pltpu.VMEM((16,), jnp.int32),
            pltpu.VMEM((16,), jnp.int32),
            pltpu.VMEM((1, _WROWS, _WCOLS), jnp.float32),
            pltpu.VMEM((1, _WROWS, _WCOLS), jnp.float32),
            pltpu.VMEM((1, _ROWS, _SIZE), jnp.float32),
            pltpu.VMEM((1, _ROWS, _SIZE), jnp.float32),
            pltpu.SemaphoreType.DMA((2,)),
            pltpu.SemaphoreType.DMA((2,)),
        ],
        compiler_params=pltpu.CompilerParams(
            use_tc_tiling_on_sc=True, needs_layout_passes=False
        ),
    )(img, ind16)


def kernel(img, pmap, ind):
    ind16 = jnp.zeros((16,), jnp.int32).at[:2].set(ind[0])
    cropped, c = _crop_call(img, ind16)
    return cropped, c


# speedup vs baseline: 4.6266x; 1.1068x over previous
import functools

import jax
import jax.numpy as jnp
from jax import lax
from jax.experimental import pallas as pl
from jax.experimental.pallas import tpu as pltpu
from jax.experimental.pallas import tpu_sc as plsc

_SIZE = 224
_NC, _NS = 2, 16
_NW = _NC * _NS
_C = 192
_CPW = _C // _NW
_NVEC = _SIZE // 16
_ROWS = (80, 80, 64)
_ROW0 = (0, 80, 160)
_WROWS = 88
_WCOLS = 384
_NCHUNK = len(_ROWS)
_NT = _CPW * _NCHUNK


def _crop_body(img, ind16, out, c_out, ind_v, c_v, ib0, ib1, ob0, ob1, isem, osem):
    wid = lax.axis_index("s") * _NC + lax.axis_index("c")
    pltpu.sync_copy(ind16, ind_v)
    iv = ind_v[...]
    x = iv[0]
    y = iv[1]
    y8 = pl.multiple_of((y // 8) * 8, 8)
    x128 = pl.multiple_of((x // 128) * 128, 128)
    ry = y - y8
    rx = x - x128

    @pl.when(wid == 0)
    def _():
        c_v[...] = iv + _SIZE // 2
        pltpu.sync_copy(c_v.at[pl.ds(0, 2)], c_out)

    ibufs = [ib0, ib1]
    obufs = [ob0, ob1]
    base = wid * _CPW

    def in_copy(t, s):
        ch = base + t // _NCHUNK
        q = t % _NCHUNK
        return pltpu.make_async_copy(
            img.at[
                pl.ds(ch, 1),
                pl.ds(y8 + _ROW0[q], _ROWS[q] + 8),
                pl.ds(x128, _WCOLS),
            ],
            ibufs[s].at[:, pl.ds(0, _ROWS[q] + 8)],
            isem.at[s],
        )

    def out_copy(t, s):
        ch = base + t // _NCHUNK
        q = t % _NCHUNK
        return pltpu.make_async_copy(
            obufs[s].at[:, pl.ds(0, _ROWS[q])],
            out.at[pl.ds(ch, 1), pl.ds(_ROW0[q], _ROWS[q])],
            osem.at[s],
        )

    iota = lax.iota(jnp.int32, 16)
    d0 = jnp.zeros((16,), jnp.int32)
    d2 = [rx + 16 * k + iota for k in range(_NVEC)]

    def compute(t, s):
        q = t % _NCHUNK
        ib = ibufs[s]
        ob = obufs[s]

        def row(j, carry):
            j0 = 2 * j
            d1a = jnp.full((16,), ry + j0, jnp.int32)
            d1b = d1a + 1
            va = [plsc.load_gather(ib, [d0, d1a, d2[k]]) for k in range(_NVEC)]
            vb = [plsc.load_gather(ib, [d0, d1b, d2[k]]) for k in range(_NVEC)]
            for k in range(_NVEC):
                ob[0, j0, pl.ds(16 * k, 16)] = va[k]
            for k in range(_NVEC):
                ob[0, j0 + 1, pl.ds(16 * k, 16)] = vb[k]
            return carry

        lax.fori_loop(0, _ROWS[q] // 2, row, 0)

    in_copy(0, 0).start()
    in_copy(1, 1).start()
    for t in range(_NT):
        s = t % 2
        in_copy(t, s).wait()
        if t >= 2:
            out_copy(t - 2, s).wait()
        compute(t, s)
        out_copy(t, s).start()
        if t + 2 < _NT:
            in_copy(t + 2, s).start()
    out_copy(_NT - 2, _NT % 2).wait()
    out_copy(_NT - 1, (_NT + 1) % 2).wait()


@jax.jit
def _crop_call(img, ind16):
    mesh = plsc.VectorSubcoreMesh(
        core_axis_name="c", subcore_axis_name="s", num_cores=_NC, num_subcores=_NS
    )
    return pl.kernel(
        _crop_body,
        out_type=[
            jax.ShapeDtypeStruct((_C, _SIZE, _SIZE), jnp.float32),
            jax.ShapeDtypeStruct((2,), jnp.int32),
        ],
        mesh=mesh,
        scratch_types=[
            pltpu.VMEM((16,), jnp.int32),
            pltpu.VMEM((16,), jnp.int32),
            pltpu.VMEM((1, _WROWS, _WCOLS), jnp.float32),
            pltpu.VMEM((1, _WROWS, _WCOLS), jnp.float32),
            pltpu.VMEM((1, 80, _SIZE), jnp.float32),
            pltpu.VMEM((1, 80, _SIZE), jnp.float32),
            pltpu.SemaphoreType.DMA((2,)),
            pltpu.SemaphoreType.DMA((2,)),
        ],
        compiler_params=pltpu.CompilerParams(
            use_tc_tiling_on_sc=True, needs_layout_passes=False
        ),
    )(img, ind16)


def kernel(img, pmap, ind):
    ind16 = jnp.zeros((16,), jnp.int32).at[:2].set(ind[0])
    cropped, c = _crop_call(img, ind16)
    return cropped, c
